# baseline (speedup 1.0000x reference)
import jax
import jax.numpy as jnp
from jax.experimental import pallas as pl


def _gcn(x, src, dst, ew, W, b):
    n = x.shape[0]
    loop = jnp.arange(n, dtype=src.dtype)
    s = jnp.concatenate([src, loop])
    d = jnp.concatenate([dst, loop])
    w = jnp.concatenate([ew, jnp.ones((n,), dtype=ew.dtype)])
    deg = jnp.zeros((n,), dtype=x.dtype).at[d].add(w)
    dinv = jnp.where(deg > 0, deg ** -0.5, 0.0)
    norm = dinv[s] * w * dinv[d]
    h = x @ W
    msg = h[s] * norm[:, None]
    out = jnp.zeros((n, h.shape[1]), dtype=x.dtype).at[d].add(msg)
    return out + b


def _bn(h, g, b, eps=1e-5):
    m = h.mean(axis=0)
    v = h.var(axis=0)
    return (h - m) / jnp.sqrt(v + eps) * g + b


def _identity_pallas(x):
    def body(x_ref, o_ref):
        o_ref[...] = x_ref[...]
    return pl.pallas_call(
        body, out_shape=jax.ShapeDtypeStruct(x.shape, x.dtype))(x)


def kernel(x, edge_index, edge_weight, batch, params):
    p = params
    src, dst = edge_index[0], edge_index[1]
    h = x
    for i in range(3):
        h = _gcn(h, src, dst, edge_weight, p['conv%d_W' % i], p['conv%d_b' % i])
        h = _bn(h, p['bn%d_g' % i], p['bn%d_b' % i])
        h = jax.nn.relu(h)
    sums = jax.ops.segment_sum(h, batch, num_segments=8)
    cnt = jax.ops.segment_sum(jnp.ones((h.shape[0], 1), h.dtype), batch, num_segments=8)
    h = sums / jnp.maximum(cnt, 1.0)
    h = jax.nn.relu(_bn(h @ p['fc1_W'] + p['fc1_b'], p['bnf1_g'], p['bnf1_b']))
    h = jax.nn.relu(_bn(h @ p['fc2_W'] + p['fc2_b'], p['bnf2_g'], p['bnf2_b']))
    return _identity_pallas(h @ p['fco_W'] + p['fco_b'])


# speedup vs baseline: 10.8270x; 10.8270x over previous
import functools

import jax
import jax.numpy as jnp
from jax import lax
from jax.experimental import pallas as pl
from jax.experimental.pallas import tpu as pltpu
from jax.experimental.pallas import tpu_sc as plsc

N = 50000
E = 800000
F_IN = 128
H = 64
HH = H // 2
G = 8
CHUNK = 128
NCHUNKS = E // CHUNK
NT = 16
BLK = 2000
NBLK = N // BLK
EPS = 1e-5

_mesh = plsc.VectorSubcoreMesh(core_axis_name="c", subcore_axis_name="s")


def _tile_range(total, t, nt=NT):
    per = total // nt
    rem = total - per * nt
    count = per + (t < rem).astype(jnp.int32)
    base = per * t + jnp.minimum(t, rem)
    return base, count


def _deg_body(edges, deg0, deg1, chunk_v, wbuf, pbuf, deg_sh):
    c = lax.axis_index("c")
    t = lax.axis_index("s")

    for i in range(64):
        pbuf[pl.ds(16 * i, 16)] = jnp.zeros((16,), jnp.float32)
    npieces = (N + 1023) // 1024
    pb, pc = _tile_range(npieces, t)

    def zero_piece(j, _):
        p = pb + j
        b = jnp.minimum(p * 1024, N - 1024)
        pltpu.sync_copy(pbuf, deg_sh.at[pl.ds(b, 1024)])
        return _

    lax.fori_loop(0, pc, zero_piece, 0)
    plsc.subcore_barrier()

    half = NCHUNKS // 2
    cb, cc = _tile_range(half, t)
    cb = cb + c * half

    def chunk_body(j, _):
        i = cb + j
        pltpu.sync_copy(edges.at[i], chunk_v)
        for g in range(CHUNK // 16):
            wbuf[pl.ds(16 * g, 16)] = lax.bitcast_convert_type(
                chunk_v[2, pl.ds(16 * g, 16)], jnp.float32)
        pltpu.sync_copy(wbuf, deg_sh.at[chunk_v.at[1]], add=True)
        return _

    lax.fori_loop(0, cc, chunk_body, 0)
    plsc.subcore_barrier()

    def rb_piece(j, carry):
        p = pb + j
        b = jnp.minimum(p * 1024, N - 1024)
        pltpu.sync_copy(deg_sh.at[pl.ds(b, 1024)], pbuf)

        @pl.when(c == 0)
        def _w0():
            pltpu.sync_copy(pbuf, deg0.at[pl.ds(b, 1024)])

        @pl.when(c == 1)
        def _w1():
            pltpu.sync_copy(pbuf, deg1.at[pl.ds(b, 1024)])
        return carry

    lax.fori_loop(0, pc, rb_piece, 0)


def _deg_call():
    return pl.kernel(
        _deg_body,
        out_type=(jax.ShapeDtypeStruct((N,), jnp.float32),
                  jax.ShapeDtypeStruct((N,), jnp.float32)),
        mesh=_mesh,
        compiler_params=pltpu.CompilerParams(use_tc_tiling_on_sc=False),
        scratch_types=[
            pltpu.VMEM((3, CHUNK), jnp.int32),
            pltpu.VMEM((CHUNK,), jnp.float32),
            pltpu.VMEM((1024,), jnp.float32),
            pltpu.VMEM_SHARED((N,), jnp.float32),
        ],
    )


def _msg_body(edges, za, zb, outa, outb, chunk_v, rows_v, sem, acc_sh):
    c = lax.axis_index("c")
    t = lax.axis_index("s")

    for r in range(CHUNK):
        for q in range(HH // 16):
            rows_v[r, pl.ds(16 * q, 16)] = jnp.zeros((16,), jnp.float32)

    npieces = (N + 127) // 128
    pb, pc = _tile_range(npieces, t)

    def zero_piece(j, _):
        p = pb + j
        b = jnp.minimum(p * 128, N - 128)
        pltpu.sync_copy(rows_v, acc_sh.at[pl.ds(b, 128)])
        return _

    lax.fori_loop(0, pc, zero_piece, 0)
    plsc.subcore_barrier()

    cb, cc = _tile_range(NCHUNKS, t)

    def chunk_body(j, carry):
        i = cb + j
        pltpu.sync_copy(edges.at[i], chunk_v)

        @pl.when(c == 0)
        def _g0():
            pltpu.async_copy(za.at[chunk_v.at[0]], rows_v, sem).wait()

        @pl.when(c == 1)
        def _g1():
            pltpu.async_copy(zb.at[chunk_v.at[0]], rows_v, sem).wait()

        for g in range(CHUNK // 16):
            wv = lax.bitcast_convert_type(chunk_v[2, pl.ds(16 * g, 16)], jnp.float32)
            for e in range(16):
                s = wv[e]
                r = 16 * g + e
                for q in range(HH // 16):
                    rows_v[r, pl.ds(16 * q, 16)] = (
                        rows_v[r, pl.ds(16 * q, 16)] * s)

        pltpu.sync_copy(rows_v, acc_sh.at[chunk_v.at[1]], add=True)
        return carry

    lax.fori_loop(0, cc, chunk_body, 0)
    plsc.subcore_barrier()

    def rb_piece(j, carry):
        p = pb + j
        b = jnp.minimum(p * 128, N - 128)
        pltpu.sync_copy(acc_sh.at[pl.ds(b, 128)], rows_v)

        @pl.when(c == 0)
        def _w0():
            pltpu.sync_copy(rows_v, outa.at[pl.ds(b, 128)])

        @pl.when(c == 1)
        def _w1():
            pltpu.sync_copy(rows_v, outb.at[pl.ds(b, 128)])
        return carry

    lax.fori_loop(0, pc, rb_piece, 0)


def _msg_call():
    return pl.kernel(
        _msg_body,
        out_type=(jax.ShapeDtypeStruct((N, HH), jnp.float32),
                  jax.ShapeDtypeStruct((N, HH), jnp.float32)),
        mesh=_mesh,
        compiler_params=pltpu.CompilerParams(use_tc_tiling_on_sc=False),
        scratch_types=[
            pltpu.VMEM((3, CHUNK), jnp.int32),
            pltpu.VMEM((CHUNK, HH), jnp.float32),
            pltpu.SemaphoreType.DMA,
            pltpu.VMEM_SHARED((N, HH), jnp.float32),
        ],
    )


def _dinv(degA, degB):
    return 1.0 / jnp.sqrt(degA + degB + 1.0)


def _tck0_body(x_ref, w_ref, dA_ref, dB_ref, za_ref, zb_ref):
    dinv = _dinv(dA_ref[...], dB_ref[...])
    z = jnp.dot(x_ref[...], w_ref[...],
                preferred_element_type=jnp.float32) * dinv
    za_ref[...] = z[:, :HH]
    zb_ref[...] = z[:, HH:]


def _tck0(x, W, degA, degB):
    return pl.pallas_call(
        _tck0_body,
        grid=(NBLK,),
        in_specs=[
            pl.BlockSpec((BLK, F_IN), lambda i: (i, 0)),
            pl.BlockSpec((F_IN, H), lambda i: (0, 0)),
            pl.BlockSpec((BLK, 1), lambda i: (i, 0)),
            pl.BlockSpec((BLK, 1), lambda i: (i, 0)),
        ],
        out_specs=[
            pl.BlockSpec((BLK, HH), lambda i: (i, 0)),
            pl.BlockSpec((BLK, HH), lambda i: (i, 0)),
        ],
        out_shape=[
            jax.ShapeDtypeStruct((N, HH), jnp.float32),
            jax.ShapeDtypeStruct((N, HH), jnp.float32),
        ],
    )(x, W, degA, degB)


def _tce_body(oa_ref, ob_ref, za_ref, zb_ref, dA_ref, dB_ref, b_ref,
              ga_ref, gb_ref, st_ref):
    dinv = _dinv(dA_ref[...], dB_ref[...])
    bias = b_ref[...]
    ga = dinv * (oa_ref[...] + za_ref[...]) + bias[:, :HH]
    gb = dinv * (ob_ref[...] + zb_ref[...]) + bias[:, HH:]
    ga_ref[...] = ga
    gb_ref[...] = gb

    upd = jnp.concatenate([
        jnp.sum(ga, axis=0, keepdims=True),
        jnp.sum(gb, axis=0, keepdims=True),
        jnp.sum(ga * ga, axis=0, keepdims=True),
        jnp.sum(gb * gb, axis=0, keepdims=True),
        jnp.zeros((4, HH), jnp.float32),
    ], axis=0)

    @pl.when(pl.program_id(0) == 0)
    def _():
        st_ref[...] = jnp.zeros((8, HH), jnp.float32)

    st_ref[...] += upd


def _tce(oa, ob, za, zb, degA, degB, bias):
    return pl.pallas_call(
        _tce_body,
        grid=(NBLK,),
        in_specs=[
            pl.BlockSpec((BLK, HH), lambda i: (i, 0)),
            pl.BlockSpec((BLK, HH), lambda i: (i, 0)),
            pl.BlockSpec((BLK, HH), lambda i: (i, 0)),
            pl.BlockSpec((BLK, HH), lambda i: (i, 0)),
            pl.BlockSpec((BLK, 1), lambda i: (i, 0)),
            pl.BlockSpec((BLK, 1), lambda i: (i, 0)),
            pl.BlockSpec((1, H), lambda i: (0, 0)),
        ],
        out_specs=[
            pl.BlockSpec((BLK, HH), lambda i: (i, 0)),
            pl.BlockSpec((BLK, HH), lambda i: (i, 0)),
            pl.BlockSpec((8, HH), lambda i: (0, 0)),
        ],
        out_shape=[
            jax.ShapeDtypeStruct((N, HH), jnp.float32),
            jax.ShapeDtypeStruct((N, HH), jnp.float32),
            jax.ShapeDtypeStruct((8, HH), jnp.float32),
        ],
    )(oa, ob, za, zb, degA, degB, bias)


def _bn_affine(ga, gb, st, gamma, beta):
    ma = st[0:1, :] * (1.0 / N)
    mb = st[1:2, :] * (1.0 / N)
    va = st[2:3, :] * (1.0 / N) - ma * ma
    vb = st[3:4, :] * (1.0 / N) - mb * mb
    ia = 1.0 / jnp.sqrt(va + EPS)
    ib = 1.0 / jnp.sqrt(vb + EPS)
    ha = jnp.maximum((ga - ma) * ia * gamma[:, :HH] + beta[:, :HH], 0.0)
    hb = jnp.maximum((gb - mb) * ib * gamma[:, HH:] + beta[:, HH:], 0.0)
    return ha, hb


def _tck_body(ga_ref, gb_ref, st_ref, gam_ref, bet_ref, w_ref, dA_ref, dB_ref,
              za_ref, zb_ref):
    ha, hb = _bn_affine(ga_ref[...], gb_ref[...], st_ref[...],
                        gam_ref[...], bet_ref[...])
    h = jnp.concatenate([ha, hb], axis=1)
    dinv = _dinv(dA_ref[...], dB_ref[...])
    z = jnp.dot(h, w_ref[...], preferred_element_type=jnp.float32) * dinv
    za_ref[...] = z[:, :HH]
    zb_ref[...] = z[:, HH:]


def _tck(ga, gb, st, gamma, beta, W, degA, degB):
    return pl.pallas_call(
        _tck_body,
        grid=(NBLK,),
        in_specs=[
            pl.BlockSpec((BLK, HH), lambda i: (i, 0)),
            pl.BlockSpec((BLK, HH), lambda i: (i, 0)),
            pl.BlockSpec((8, HH), lambda i: (0, 0)),
            pl.BlockSpec((1, H), lambda i: (0, 0)),
            pl.BlockSpec((1, H), lambda i: (0, 0)),
            pl.BlockSpec((H, H), lambda i: (0, 0)),
            pl.BlockSpec((BLK, 1), lambda i: (i, 0)),
            pl.BlockSpec((BLK, 1), lambda i: (i, 0)),
        ],
        out_specs=[
            pl.BlockSpec((BLK, HH), lambda i: (i, 0)),
            pl.BlockSpec((BLK, HH), lambda i: (i, 0)),
        ],
        out_shape=[
            jax.ShapeDtypeStruct((N, HH), jnp.float32),
            jax.ShapeDtypeStruct((N, HH), jnp.float32),
        ],
    )(ga, gb, st, gamma, beta, W, degA, degB)


def _head_body(ga_ref, gb_ref, st_ref, gam_ref, bet_ref, batch_ref,
               fc1w_ref, fc1b_ref, g1_ref, b1_ref,
               fc2w_ref, fc2b_ref, g2_ref, b2_ref,
               fcow_ref, fcob_ref, out_ref, acc_ref):
    ha, hb = _bn_affine(ga_ref[...], gb_ref[...], st_ref[...],
                        gam_ref[...], bet_ref[...])
    ones = jnp.ones((BLK, 1), jnp.float32)
    zeros = jnp.zeros((BLK, 128 - H - 1), jnp.float32)
    h_aug = jnp.concatenate([ha, hb, ones, zeros], axis=1)
    gids = lax.broadcasted_iota(jnp.int32, (1, G), 1)
    onehot = (batch_ref[...] == gids).astype(jnp.float32)
    psum = lax.dot_general(onehot, h_aug, (((0,), (0,)), ((), ())),
                           preferred_element_type=jnp.float32,
                           precision=lax.Precision.HIGHEST)

    @pl.when(pl.program_id(0) == 0)
    def _():
        acc_ref[...] = jnp.zeros((G, 128), jnp.float32)

    acc_ref[...] += psum

    @pl.when(pl.program_id(0) == NBLK - 1)
    def _():
        acc = acc_ref[...]
        pooled = acc[:, :H] / jnp.maximum(acc[:, H:H + 1], 1.0)

        def bn_small(v, g, b):
            m = jnp.mean(v, axis=0, keepdims=True)
            var = jnp.mean((v - m) * (v - m), axis=0, keepdims=True)
            return (v - m) / jnp.sqrt(var + EPS) * g + b

        t1 = jnp.dot(pooled, fc1w_ref[...],
                     preferred_element_type=jnp.float32) + fc1b_ref[...]
        t1 = jnp.maximum(bn_small(t1, g1_ref[...], b1_ref[...]), 0.0)
        t2 = jnp.dot(t1, fc2w_ref[...],
                     preferred_element_type=jnp.float32) + fc2b_ref[...]
        t2 = jnp.maximum(bn_small(t2, g2_ref[...], b2_ref[...]), 0.0)
        out_ref[...] = jnp.dot(t2, fcow_ref[...],
                               preferred_element_type=jnp.float32) + fcob_ref[...]


def _head(ga, gb, st, gamma, beta, batch2d, p):
    full = lambda shape: pl.BlockSpec(shape, lambda i: tuple(0 for _ in shape))
    return pl.pallas_call(
        _head_body,
        grid=(NBLK,),
        in_specs=[
            pl.BlockSpec((BLK, HH), lambda i: (i, 0)),
            pl.BlockSpec((BLK, HH), lambda i: (i, 0)),
            full((8, HH)),
            full((1, H)),
            full((1, H)),
            pl.BlockSpec((BLK, 1), lambda i: (i, 0)),
            full((H, H)),
            full((1, H)),
            full((1, H)),
            full((1, H)),
            full((H, HH)),
            full((1, HH)),
            full((1, HH)),
            full((1, HH)),
            full((HH, 2)),
            full((1, 2)),
        ],
        out_specs=pl.BlockSpec((G, 2), lambda i: (0, 0)),
        out_shape=jax.ShapeDtypeStruct((G, 2), jnp.float32),
        scratch_shapes=[pltpu.VMEM((G, 128), jnp.float32)],
    )(ga, gb, st, gamma, beta, batch2d,
      p['fc1_W'], p['fc1_b'].reshape(1, H),
      p['bnf1_g'].reshape(1, H), p['bnf1_b'].reshape(1, H),
      p['fc2_W'], p['fc2_b'].reshape(1, HH),
      p['bnf2_g'].reshape(1, HH), p['bnf2_b'].reshape(1, HH),
      p['fco_W'], p['fco_b'].reshape(1, 2))


def kernel(x, edge_index, edge_weight, batch, params):
    p = params
    src = edge_index[0]
    dst = edge_index[1]
    wbits = lax.bitcast_convert_type(edge_weight, jnp.int32)
    edges = jnp.stack([
        src.reshape(NCHUNKS, CHUNK),
        dst.reshape(NCHUNKS, CHUNK),
        wbits.reshape(NCHUNKS, CHUNK),
    ], axis=1)

    deg0, deg1 = _deg_call()(edges)
    degA = deg0.reshape(N, 1)
    degB = deg1.reshape(N, 1)

    za, zb = _tck0(x, p['conv0_W'], degA, degB)
    oa, ob = _msg_call()(edges, za, zb)
    ga, gb, st = _tce(oa, ob, za, zb, degA, degB, p['conv0_b'].reshape(1, H))

    for i in (1, 2):
        za, zb = _tck(ga, gb, st,
                      p['bn%d_g' % (i - 1)].reshape(1, H),
                      p['bn%d_b' % (i - 1)].reshape(1, H),
                      p['conv%d_W' % i], degA, degB)
        oa, ob = _msg_call()(edges, za, zb)
        ga, gb, st = _tce(oa, ob, za, zb, degA, degB,
                          p['conv%d_b' % i].reshape(1, H))

    return _head(ga, gb, st,
                 p['bn2_g'].reshape(1, H), p['bn2_b'].reshape(1, H),
                 batch.reshape(N, 1), p)


# speedup vs baseline: 18.4394x; 1.7031x over previous
import functools

import jax
import jax.numpy as jnp
from jax import lax
from jax.experimental import pallas as pl
from jax.experimental.pallas import tpu as pltpu
from jax.experimental.pallas import tpu_sc as plsc

N = 50000
E = 800000
F_IN = 128
H = 64
HH = H // 2
G = 8
CHUNK = 128
NCHUNKS = E // CHUNK
NT = 16
BLK = 2000
NBLK = N // BLK
EPS = 1e-5

_mesh = plsc.VectorSubcoreMesh(core_axis_name="c", subcore_axis_name="s")


def _tile_range(total, t, nt=NT):
    per = total // nt
    rem = total - per * nt
    count = per + (t < rem).astype(jnp.int32)
    base = per * t + jnp.minimum(t, rem)
    return base, count


def _deg_body(edges, deg0, deg1, chunk_v, wbuf, pbuf, deg_sh):
    c = lax.axis_index("c")
    t = lax.axis_index("s")

    for i in range(64):
        pbuf[pl.ds(16 * i, 16)] = jnp.zeros((16,), jnp.float32)
    npieces = (N + 1023) // 1024
    pb, pc = _tile_range(npieces, t)

    def zero_piece(j, _):
        p = pb + j
        b = jnp.minimum(p * 1024, N - 1024)
        pltpu.sync_copy(pbuf, deg_sh.at[pl.ds(b, 1024)])
        return _

    lax.fori_loop(0, pc, zero_piece, 0)
    plsc.subcore_barrier()

    half = NCHUNKS // 2
    cb, cc = _tile_range(half, t)
    cb = cb + c * half

    def chunk_body(j, _):
        i = cb + j
        pltpu.sync_copy(edges.at[i], chunk_v)
        for g in range(CHUNK // 16):
            wbuf[pl.ds(16 * g, 16)] = lax.bitcast_convert_type(
                chunk_v[2, pl.ds(16 * g, 16)], jnp.float32)
        pltpu.sync_copy(wbuf, deg_sh.at[chunk_v.at[1]], add=True)
        return _

    lax.fori_loop(0, cc, chunk_body, 0)
    plsc.subcore_barrier()

    def rb_piece(j, carry):
        p = pb + j
        b = jnp.minimum(p * 1024, N - 1024)
        pltpu.sync_copy(deg_sh.at[pl.ds(b, 1024)], pbuf)

        @pl.when(c == 0)
        def _w0():
            pltpu.sync_copy(pbuf, deg0.at[pl.ds(b, 1024)])

        @pl.when(c == 1)
        def _w1():
            pltpu.sync_copy(pbuf, deg1.at[pl.ds(b, 1024)])
        return carry

    lax.fori_loop(0, pc, rb_piece, 0)


def _deg_call():
    return pl.kernel(
        _deg_body,
        out_type=(jax.ShapeDtypeStruct((N,), jnp.float32),
                  jax.ShapeDtypeStruct((N,), jnp.float32)),
        mesh=_mesh,
        compiler_params=pltpu.CompilerParams(use_tc_tiling_on_sc=False),
        scratch_types=[
            pltpu.VMEM((3, CHUNK), jnp.int32),
            pltpu.VMEM((CHUNK,), jnp.float32),
            pltpu.VMEM((1024,), jnp.float32),
            pltpu.VMEM_SHARED((N,), jnp.float32),
        ],
    )


def _msg_body(edges, za, zb, outa, outb,
              e0, e1, e2, r0, r1, r2,
              se0, se1, se2, sg0, sg1, sg2, ss0, ss1, ss2, acc_sh):
    c = lax.axis_index("c")
    t = lax.axis_index("s")
    ebufs = (e0, e1, e2)
    rbufs = (r0, r1, r2)
    esems = (se0, se1, se2)
    gsems = (sg0, sg1, sg2)
    ssems = (ss0, ss1, ss2)

    for r in range(CHUNK):
        for q in range(HH // 16):
            r0[r, pl.ds(16 * q, 16)] = jnp.zeros((16,), jnp.float32)

    npieces = (N + 127) // 128
    pb, pc = _tile_range(npieces, t)

    def zero_piece(j, _):
        p = pb + j
        b = jnp.minimum(p * 128, N - 128)
        pltpu.sync_copy(r0, acc_sh.at[pl.ds(b, 128)])
        return _

    lax.fori_loop(0, pc, zero_piece, 0)
    plsc.subcore_barrier()

    cb, cc = _tile_range(NCHUNKS, t)

    def load_edges(k, j):
        pltpu.async_copy(edges.at[cb + j], ebufs[k], esems[k])

    def wait_edges(k, j):
        pltpu.make_async_copy(edges.at[cb + j], ebufs[k], esems[k]).wait()

    def issue_gather(k):
        @pl.when(c == 0)
        def _g0():
            pltpu.async_copy(za.at[ebufs[k].at[0]], rbufs[k], gsems[k])

        @pl.when(c == 1)
        def _g1():
            pltpu.async_copy(zb.at[ebufs[k].at[0]], rbufs[k], gsems[k])

    def wait_gather(k):
        pltpu.make_async_copy(za.at[ebufs[k].at[0]], rbufs[k], gsems[k]).wait()

    def scale(k):
        eb = ebufs[k]
        rb = rbufs[k]
        for g in range(CHUNK // 16):
            wv = lax.bitcast_convert_type(eb[2, pl.ds(16 * g, 16)], jnp.float32)
            for e in range(16):
                s = wv[e]
                r = 16 * g + e
                for q in range(HH // 16):
                    rb[r, pl.ds(16 * q, 16)] = rb[r, pl.ds(16 * q, 16)] * s

    def issue_scatter(k):
        pltpu.async_copy(rbufs[k], acc_sh.at[ebufs[k].at[1]], ssems[k],
                         add=True)

    def wait_scatter(k):
        pltpu.make_async_copy(rbufs[k], acc_sh.at[ebufs[k].at[1]],
                              ssems[k]).wait()

    for k in range(3):
        load_edges(k, k)
    for k in range(3):
        wait_edges(k, k)
        issue_gather(k)

    ntriples = (cc + 2) // 3

    def triple(q, carry):
        for k in range(3):
            j = 3 * q + k

            @pl.when(j < cc)
            def _c():
                wait_gather(k)
                scale(k)
                issue_scatter(k)
        for k in range(3):
            j = 3 * q + 3 + k

            @pl.when(j < cc)
            def _a():
                wait_scatter(k)
                load_edges(k, j)
        for k in range(3):
            j = 3 * q + 3 + k

            @pl.when(j < cc)
            def _b():
                wait_edges(k, j)
                issue_gather(k)
        return carry

    lax.fori_loop(0, ntriples, triple, 0)
    for k in range(3):
        wait_scatter(k)
    plsc.subcore_barrier()

    def rb_piece(j, carry):
        p = pb + j
        b = jnp.minimum(p * 128, N - 128)
        pltpu.sync_copy(acc_sh.at[pl.ds(b, 128)], r0)

        @pl.when(c == 0)
        def _w0():
            pltpu.sync_copy(r0, outa.at[pl.ds(b, 128)])

        @pl.when(c == 1)
        def _w1():
            pltpu.sync_copy(r0, outb.at[pl.ds(b, 128)])
        return carry

    lax.fori_loop(0, pc, rb_piece, 0)


def _msg_call():
    return pl.kernel(
        _msg_body,
        out_type=(jax.ShapeDtypeStruct((N, HH), jnp.float32),
                  jax.ShapeDtypeStruct((N, HH), jnp.float32)),
        mesh=_mesh,
        compiler_params=pltpu.CompilerParams(use_tc_tiling_on_sc=False),
        scratch_types=[
            pltpu.VMEM((3, CHUNK), jnp.int32),
            pltpu.VMEM((3, CHUNK), jnp.int32),
            pltpu.VMEM((3, CHUNK), jnp.int32),
            pltpu.VMEM((CHUNK, HH), jnp.float32),
            pltpu.VMEM((CHUNK, HH), jnp.float32),
            pltpu.VMEM((CHUNK, HH), jnp.float32),
            pltpu.SemaphoreType.DMA,
            pltpu.SemaphoreType.DMA,
            pltpu.SemaphoreType.DMA,
            pltpu.SemaphoreType.DMA,
            pltpu.SemaphoreType.DMA,
            pltpu.SemaphoreType.DMA,
            pltpu.SemaphoreType.DMA,
            pltpu.SemaphoreType.DMA,
            pltpu.SemaphoreType.DMA,
            pltpu.VMEM_SHARED((N, HH), jnp.float32),
        ],
    )


def _dinv(degA, degB):
    return 1.0 / jnp.sqrt(degA + degB + 1.0)


def _tck0_body(x_ref, w_ref, dA_ref, dB_ref, za_ref, zb_ref):
    dinv = _dinv(dA_ref[...], dB_ref[...])
    z = jnp.dot(x_ref[...], w_ref[...],
                preferred_element_type=jnp.float32) * dinv
    za_ref[...] = z[:, :HH]
    zb_ref[...] = z[:, HH:]


def _tck0(x, W, degA, degB):
    return pl.pallas_call(
        _tck0_body,
        grid=(NBLK,),
        in_specs=[
            pl.BlockSpec((BLK, F_IN), lambda i: (i, 0)),
            pl.BlockSpec((F_IN, H), lambda i: (0, 0)),
            pl.BlockSpec((BLK, 1), lambda i: (i, 0)),
            pl.BlockSpec((BLK, 1), lambda i: (i, 0)),
        ],
        out_specs=[
            pl.BlockSpec((BLK, HH), lambda i: (i, 0)),
            pl.BlockSpec((BLK, HH), lambda i: (i, 0)),
        ],
        out_shape=[
            jax.ShapeDtypeStruct((N, HH), jnp.float32),
            jax.ShapeDtypeStruct((N, HH), jnp.float32),
        ],
    )(x, W, degA, degB)


def _tce_body(oa_ref, ob_ref, za_ref, zb_ref, dA_ref, dB_ref, b_ref,
              ga_ref, gb_ref, st_ref):
    dinv = _dinv(dA_ref[...], dB_ref[...])
    bias = b_ref[...]
    ga = dinv * (oa_ref[...] + za_ref[...]) + bias[:, :HH]
    gb = dinv * (ob_ref[...] + zb_ref[...]) + bias[:, HH:]
    ga_ref[...] = ga
    gb_ref[...] = gb

    upd = jnp.concatenate([
        jnp.sum(ga, axis=0, keepdims=True),
        jnp.sum(gb, axis=0, keepdims=True),
        jnp.sum(ga * ga, axis=0, keepdims=True),
        jnp.sum(gb * gb, axis=0, keepdims=True),
        jnp.zeros((4, HH), jnp.float32),
    ], axis=0)

    @pl.when(pl.program_id(0) == 0)
    def _():
        st_ref[...] = jnp.zeros((8, HH), jnp.float32)

    st_ref[...] += upd


def _tce(oa, ob, za, zb, degA, degB, bias):
    return pl.pallas_call(
        _tce_body,
        grid=(NBLK,),
        in_specs=[
            pl.BlockSpec((BLK, HH), lambda i: (i, 0)),
            pl.BlockSpec((BLK, HH), lambda i: (i, 0)),
            pl.BlockSpec((BLK, HH), lambda i: (i, 0)),
            pl.BlockSpec((BLK, HH), lambda i: (i, 0)),
            pl.BlockSpec((BLK, 1), lambda i: (i, 0)),
            pl.BlockSpec((BLK, 1), lambda i: (i, 0)),
            pl.BlockSpec((1, H), lambda i: (0, 0)),
        ],
        out_specs=[
            pl.BlockSpec((BLK, HH), lambda i: (i, 0)),
            pl.BlockSpec((BLK, HH), lambda i: (i, 0)),
            pl.BlockSpec((8, HH), lambda i: (0, 0)),
        ],
        out_shape=[
            jax.ShapeDtypeStruct((N, HH), jnp.float32),
            jax.ShapeDtypeStruct((N, HH), jnp.float32),
            jax.ShapeDtypeStruct((8, HH), jnp.float32),
        ],
    )(oa, ob, za, zb, degA, degB, bias)


def _bn_affine(ga, gb, st, gamma, beta):
    ma = st[0:1, :] * (1.0 / N)
    mb = st[1:2, :] * (1.0 / N)
    va = st[2:3, :] * (1.0 / N) - ma * ma
    vb = st[3:4, :] * (1.0 / N) - mb * mb
    ia = 1.0 / jnp.sqrt(va + EPS)
    ib = 1.0 / jnp.sqrt(vb + EPS)
    ha = jnp.maximum((ga - ma) * ia * gamma[:, :HH] + beta[:, :HH], 0.0)
    hb = jnp.maximum((gb - mb) * ib * gamma[:, HH:] + beta[:, HH:], 0.0)
    return ha, hb


def _tck_body(ga_ref, gb_ref, st_ref, gam_ref, bet_ref, w_ref, dA_ref, dB_ref,
              za_ref, zb_ref):
    ha, hb = _bn_affine(ga_ref[...], gb_ref[...], st_ref[...],
                        gam_ref[...], bet_ref[...])
    h = jnp.concatenate([ha, hb], axis=1)
    dinv = _dinv(dA_ref[...], dB_ref[...])
    z = jnp.dot(h, w_ref[...], preferred_element_type=jnp.float32) * dinv
    za_ref[...] = z[:, :HH]
    zb_ref[...] = z[:, HH:]


def _tck(ga, gb, st, gamma, beta, W, degA, degB):
    return pl.pallas_call(
        _tck_body,
        grid=(NBLK,),
        in_specs=[
            pl.BlockSpec((BLK, HH), lambda i: (i, 0)),
            pl.BlockSpec((BLK, HH), lambda i: (i, 0)),
            pl.BlockSpec((8, HH), lambda i: (0, 0)),
            pl.BlockSpec((1, H), lambda i: (0, 0)),
            pl.BlockSpec((1, H), lambda i: (0, 0)),
            pl.BlockSpec((H, H), lambda i: (0, 0)),
            pl.BlockSpec((BLK, 1), lambda i: (i, 0)),
            pl.BlockSpec((BLK, 1), lambda i: (i, 0)),
        ],
        out_specs=[
            pl.BlockSpec((BLK, HH), lambda i: (i, 0)),
            pl.BlockSpec((BLK, HH), lambda i: (i, 0)),
        ],
        out_shape=[
            jax.ShapeDtypeStruct((N, HH), jnp.float32),
            jax.ShapeDtypeStruct((N, HH), jnp.float32),
        ],
    )(ga, gb, st, gamma, beta, W, degA, degB)


def _head_body(ga_ref, gb_ref, st_ref, gam_ref, bet_ref, batch_ref,
               fc1w_ref, fc1b_ref, g1_ref, b1_ref,
               fc2w_ref, fc2b_ref, g2_ref, b2_ref,
               fcow_ref, fcob_ref, out_ref, acc_ref):
    ha, hb = _bn_affine(ga_ref[...], gb_ref[...], st_ref[...],
                        gam_ref[...], bet_ref[...])
    ones = jnp.ones((BLK, 1), jnp.float32)
    zeros = jnp.zeros((BLK, 128 - H - 1), jnp.float32)
    h_aug = jnp.concatenate([ha, hb, ones, zeros], axis=1)
    gids = lax.broadcasted_iota(jnp.int32, (1, G), 1)
    onehot = (batch_ref[...] == gids).astype(jnp.float32)
    psum = lax.dot_general(onehot, h_aug, (((0,), (0,)), ((), ())),
                           preferred_element_type=jnp.float32,
                           precision=lax.Precision.HIGHEST)

    @pl.when(pl.program_id(0) == 0)
    def _():
        acc_ref[...] = jnp.zeros((G, 128), jnp.float32)

    acc_ref[...] += psum

    @pl.when(pl.program_id(0) == NBLK - 1)
    def _():
        acc = acc_ref[...]
        pooled = acc[:, :H] / jnp.maximum(acc[:, H:H + 1], 1.0)

        def bn_small(v, g, b):
            m = jnp.mean(v, axis=0, keepdims=True)
            var = jnp.mean((v - m) * (v - m), axis=0, keepdims=True)
            return (v - m) / jnp.sqrt(var + EPS) * g + b

        t1 = jnp.dot(pooled, fc1w_ref[...],
                     preferred_element_type=jnp.float32) + fc1b_ref[...]
        t1 = jnp.maximum(bn_small(t1, g1_ref[...], b1_ref[...]), 0.0)
        t2 = jnp.dot(t1, fc2w_ref[...],
                     preferred_element_type=jnp.float32) + fc2b_ref[...]
        t2 = jnp.maximum(bn_small(t2, g2_ref[...], b2_ref[...]), 0.0)
        out_ref[...] = jnp.dot(t2, fcow_ref[...],
                               preferred_element_type=jnp.float32) + fcob_ref[...]


def _head(ga, gb, st, gamma, beta, batch2d, p):
    full = lambda shape: pl.BlockSpec(shape, lambda i: tuple(0 for _ in shape))
    return pl.pallas_call(
        _head_body,
        grid=(NBLK,),
        in_specs=[
            pl.BlockSpec((BLK, HH), lambda i: (i, 0)),
            pl.BlockSpec((BLK, HH), lambda i: (i, 0)),
            full((8, HH)),
            full((1, H)),
            full((1, H)),
            pl.BlockSpec((BLK, 1), lambda i: (i, 0)),
            full((H, H)),
            full((1, H)),
            full((1, H)),
            full((1, H)),
            full((H, HH)),
            full((1, HH)),
            full((1, HH)),
            full((1, HH)),
            full((HH, 2)),
            full((1, 2)),
        ],
        out_specs=pl.BlockSpec((G, 2), lambda i: (0, 0)),
        out_shape=jax.ShapeDtypeStruct((G, 2), jnp.float32),
        scratch_shapes=[pltpu.VMEM((G, 128), jnp.float32)],
    )(ga, gb, st, gamma, beta, batch2d,
      p['fc1_W'], p['fc1_b'].reshape(1, H),
      p['bnf1_g'].reshape(1, H), p['bnf1_b'].reshape(1, H),
      p['fc2_W'], p['fc2_b'].reshape(1, HH),
      p['bnf2_g'].reshape(1, HH), p['bnf2_b'].reshape(1, HH),
      p['fco_W'], p['fco_b'].reshape(1, 2))


def kernel(x, edge_index, edge_weight, batch, params):
    p = params
    src = edge_index[0]
    dst = edge_index[1]
    wbits = lax.bitcast_convert_type(edge_weight, jnp.int32)
    edges = jnp.stack([
        src.reshape(NCHUNKS, CHUNK),
        dst.reshape(NCHUNKS, CHUNK),
        wbits.reshape(NCHUNKS, CHUNK),
    ], axis=1)

    deg0, deg1 = _deg_call()(edges)
    degA = deg0.reshape(N, 1)
    degB = deg1.reshape(N, 1)

    za, zb = _tck0(x, p['conv0_W'], degA, degB)
    oa, ob = _msg_call()(edges, za, zb)
    ga, gb, st = _tce(oa, ob, za, zb, degA, degB, p['conv0_b'].reshape(1, H))

    for i in (1, 2):
        za, zb = _tck(ga, gb, st,
                      p['bn%d_g' % (i - 1)].reshape(1, H),
                      p['bn%d_b' % (i - 1)].reshape(1, H),
                      p['conv%d_W' % i], degA, degB)
        oa, ob = _msg_call()(edges, za, zb)
        ga, gb, st = _tce(oa, ob, za, zb, degA, degB,
                          p['conv%d_b' % i].reshape(1, H))

    return _head(ga, gb, st,
                 p['bn2_g'].reshape(1, H), p['bn2_b'].reshape(1, H),
                 batch.reshape(N, 1), p)


# speedup vs baseline: 19.3495x; 1.0494x over previous
import functools

import jax
import jax.numpy as jnp
from jax import lax
from jax.experimental import pallas as pl
from jax.experimental.pallas import tpu as pltpu
from jax.experimental.pallas import tpu_sc as plsc

N = 50000
E = 800000
F_IN = 128
H = 64
HH = H // 2
G = 8
CHUNK = 128
NCHUNKS = E // CHUNK
NT = 16
BLK = 2000
NBLK = N // BLK
EPS = 1e-5

_mesh = plsc.VectorSubcoreMesh(core_axis_name="c", subcore_axis_name="s")


def _tile_range(total, t, nt=NT):
    per = total // nt
    rem = total - per * nt
    count = per + (t < rem).astype(jnp.int32)
    base = per * t + jnp.minimum(t, rem)
    return base, count


def _deg_body(edges, deg0, deg1, ce0, ce1, ce2, w0, w1, w2, pbuf,
              se0, se1, se2, ss0, ss1, ss2, deg_sh):
    c = lax.axis_index("c")
    t = lax.axis_index("s")

    for i in range(64):
        pbuf[pl.ds(16 * i, 16)] = jnp.zeros((16,), jnp.float32)
    npieces = (N + 1023) // 1024
    pb, pc = _tile_range(npieces, t)

    def zero_piece(j, _):
        p = pb + j
        b = jnp.minimum(p * 1024, N - 1024)
        pltpu.sync_copy(pbuf, deg_sh.at[pl.ds(b, 1024)])
        return _

    lax.fori_loop(0, pc, zero_piece, 0)
    plsc.subcore_barrier()

    half = NCHUNKS // 2
    cb, cc = _tile_range(half, t)
    cb = cb + c * half
    ebufs = (ce0, ce1, ce2)
    wbufs = (w0, w1, w2)
    esems = (se0, se1, se2)
    ssems = (ss0, ss1, ss2)

    def load_edges(k, j):
        pltpu.async_copy(edges.at[cb + j], ebufs[k], esems[k])

    def wait_edges(k, j):
        pltpu.make_async_copy(edges.at[cb + j], ebufs[k], esems[k]).wait()

    def do_scatter(k):
        eb = ebufs[k]
        wb = wbufs[k]
        for g in range(CHUNK // 16):
            wb[pl.ds(16 * g, 16)] = lax.bitcast_convert_type(
                eb[2, pl.ds(16 * g, 16)], jnp.float32)
        pltpu.async_copy(wb, deg_sh.at[eb.at[1]], ssems[k], add=True)

    def wait_scatter(k):
        pltpu.make_async_copy(wbufs[k], deg_sh.at[ebufs[k].at[1]],
                              ssems[k]).wait()

    for k in range(3):
        load_edges(k, k)

    ntriples = (cc + 2) // 3

    def triple(q, carry):
        for k in range(3):
            j = 3 * q + k

            @pl.when(j < cc)
            def _c():
                wait_edges(k, j)
                do_scatter(k)
        for k in range(3):
            j = 3 * q + 3 + k

            @pl.when(j < cc)
            def _a():
                wait_scatter(k)
                load_edges(k, j)
        return carry

    lax.fori_loop(0, ntriples, triple, 0)
    for k in range(3):
        wait_scatter(k)
    plsc.subcore_barrier()

    def rb_piece(j, carry):
        p = pb + j
        b = jnp.minimum(p * 1024, N - 1024)
        pltpu.sync_copy(deg_sh.at[pl.ds(b, 1024)], pbuf)

        @pl.when(c == 0)
        def _w0():
            pltpu.sync_copy(pbuf, deg0.at[pl.ds(b, 1024)])

        @pl.when(c == 1)
        def _w1():
            pltpu.sync_copy(pbuf, deg1.at[pl.ds(b, 1024)])
        return carry

    lax.fori_loop(0, pc, rb_piece, 0)


def _deg_call():
    return pl.kernel(
        _deg_body,
        out_type=(jax.ShapeDtypeStruct((N,), jnp.float32),
                  jax.ShapeDtypeStruct((N,), jnp.float32)),
        mesh=_mesh,
        compiler_params=pltpu.CompilerParams(use_tc_tiling_on_sc=False),
        scratch_types=[
            pltpu.VMEM((3, CHUNK), jnp.int32),
            pltpu.VMEM((3, CHUNK), jnp.int32),
            pltpu.VMEM((3, CHUNK), jnp.int32),
            pltpu.VMEM((CHUNK,), jnp.float32),
            pltpu.VMEM((CHUNK,), jnp.float32),
            pltpu.VMEM((CHUNK,), jnp.float32),
            pltpu.VMEM((1024,), jnp.float32),
            pltpu.SemaphoreType.DMA,
            pltpu.SemaphoreType.DMA,
            pltpu.SemaphoreType.DMA,
            pltpu.SemaphoreType.DMA,
            pltpu.SemaphoreType.DMA,
            pltpu.SemaphoreType.DMA,
            pltpu.VMEM_SHARED((N,), jnp.float32),
        ],
    )


def _msg_body(edges, za, zb, outa, outb,
              e0, e1, e2, r0, r1, r2,
              se0, se1, se2, sg0, sg1, sg2, ss0, ss1, ss2, acc_sh):
    c = lax.axis_index("c")
    t = lax.axis_index("s")
    ebufs = (e0, e1, e2)
    rbufs = (r0, r1, r2)
    esems = (se0, se1, se2)
    gsems = (sg0, sg1, sg2)
    ssems = (ss0, ss1, ss2)

    for r in range(CHUNK):
        for q in range(HH // 16):
            r0[r, pl.ds(16 * q, 16)] = jnp.zeros((16,), jnp.float32)

    npieces = (N + 127) // 128
    pb, pc = _tile_range(npieces, t)

    def zero_piece(j, _):
        p = pb + j
        b = jnp.minimum(p * 128, N - 128)
        pltpu.sync_copy(r0, acc_sh.at[pl.ds(b, 128)])
        return _

    lax.fori_loop(0, pc, zero_piece, 0)
    plsc.subcore_barrier()

    cb, cc = _tile_range(NCHUNKS, t)

    def load_edges(k, j):
        pltpu.async_copy(edges.at[cb + j], ebufs[k], esems[k])

    def wait_edges(k, j):
        pltpu.make_async_copy(edges.at[cb + j], ebufs[k], esems[k]).wait()

    def issue_gather(k):
        @pl.when(c == 0)
        def _g0():
            pltpu.async_copy(za.at[ebufs[k].at[0]], rbufs[k], gsems[k])

        @pl.when(c == 1)
        def _g1():
            pltpu.async_copy(zb.at[ebufs[k].at[0]], rbufs[k], gsems[k])

    def wait_gather(k):
        pltpu.make_async_copy(za.at[ebufs[k].at[0]], rbufs[k], gsems[k]).wait()

    def scale(k):
        eb = ebufs[k]
        rb = rbufs[k]
        for g in range(CHUNK // 16):
            wv = lax.bitcast_convert_type(eb[2, pl.ds(16 * g, 16)], jnp.float32)
            for e in range(16):
                s = wv[e]
                r = 16 * g + e
                for q in range(HH // 16):
                    rb[r, pl.ds(16 * q, 16)] = rb[r, pl.ds(16 * q, 16)] * s

    def issue_scatter(k):
        pltpu.async_copy(rbufs[k], acc_sh.at[ebufs[k].at[1]], ssems[k],
                         add=True)

    def wait_scatter(k):
        pltpu.make_async_copy(rbufs[k], acc_sh.at[ebufs[k].at[1]],
                              ssems[k]).wait()

    for k in range(3):
        load_edges(k, k)
    for k in range(3):
        wait_edges(k, k)
        issue_gather(k)

    ntriples = (cc + 2) // 3

    def triple(q, carry):
        for k in range(3):
            j = 3 * q + k

            @pl.when(j < cc)
            def _c():
                wait_gather(k)
                scale(k)
                issue_scatter(k)
        for k in range(3):
            j = 3 * q + 3 + k

            @pl.when(j < cc)
            def _a():
                wait_scatter(k)
                load_edges(k, j)
        for k in range(3):
            j = 3 * q + 3 + k

            @pl.when(j < cc)
            def _b():
                wait_edges(k, j)
                issue_gather(k)
        return carry

    lax.fori_loop(0, ntriples, triple, 0)
    for k in range(3):
        wait_scatter(k)
    plsc.subcore_barrier()

    def rb_piece(j, carry):
        p = pb + j
        b = jnp.minimum(p * 128, N - 128)
        pltpu.sync_copy(acc_sh.at[pl.ds(b, 128)], r0)

        @pl.when(c == 0)
        def _w0():
            pltpu.sync_copy(r0, outa.at[pl.ds(b, 128)])

        @pl.when(c == 1)
        def _w1():
            pltpu.sync_copy(r0, outb.at[pl.ds(b, 128)])
        return carry

    lax.fori_loop(0, pc, rb_piece, 0)


def _msg_call():
    return pl.kernel(
        _msg_body,
        out_type=(jax.ShapeDtypeStruct((N, HH), jnp.float32),
                  jax.ShapeDtypeStruct((N, HH), jnp.float32)),
        mesh=_mesh,
        compiler_params=pltpu.CompilerParams(use_tc_tiling_on_sc=False),
        scratch_types=[
            pltpu.VMEM((3, CHUNK), jnp.int32),
            pltpu.VMEM((3, CHUNK), jnp.int32),
            pltpu.VMEM((3, CHUNK), jnp.int32),
            pltpu.VMEM((CHUNK, HH), jnp.float32),
            pltpu.VMEM((CHUNK, HH), jnp.float32),
            pltpu.VMEM((CHUNK, HH), jnp.float32),
            pltpu.SemaphoreType.DMA,
            pltpu.SemaphoreType.DMA,
            pltpu.SemaphoreType.DMA,
            pltpu.SemaphoreType.DMA,
            pltpu.SemaphoreType.DMA,
            pltpu.SemaphoreType.DMA,
            pltpu.SemaphoreType.DMA,
            pltpu.SemaphoreType.DMA,
            pltpu.SemaphoreType.DMA,
            pltpu.VMEM_SHARED((N, HH), jnp.float32),
        ],
    )


def _dinv(degA, degB):
    return 1.0 / jnp.sqrt(degA + degB + 1.0)


def _tck0_body(x_ref, w_ref, dA_ref, dB_ref, za_ref, zb_ref):
    dinv = _dinv(dA_ref[...], dB_ref[...])
    z = jnp.dot(x_ref[...], w_ref[...],
                preferred_element_type=jnp.float32) * dinv
    za_ref[...] = z[:, :HH]
    zb_ref[...] = z[:, HH:]


def _tck0(x, W, degA, degB):
    return pl.pallas_call(
        _tck0_body,
        grid=(NBLK,),
        in_specs=[
            pl.BlockSpec((BLK, F_IN), lambda i: (i, 0)),
            pl.BlockSpec((F_IN, H), lambda i: (0, 0)),
            pl.BlockSpec((BLK, 1), lambda i: (i, 0)),
            pl.BlockSpec((BLK, 1), lambda i: (i, 0)),
        ],
        out_specs=[
            pl.BlockSpec((BLK, HH), lambda i: (i, 0)),
            pl.BlockSpec((BLK, HH), lambda i: (i, 0)),
        ],
        out_shape=[
            jax.ShapeDtypeStruct((N, HH), jnp.float32),
            jax.ShapeDtypeStruct((N, HH), jnp.float32),
        ],
    )(x, W, degA, degB)


def _tce_body(oa_ref, ob_ref, za_ref, zb_ref, dA_ref, dB_ref, b_ref,
              ga_ref, gb_ref, st_ref):
    dinv = _dinv(dA_ref[...], dB_ref[...])
    bias = b_ref[...]
    ga = dinv * (oa_ref[...] + za_ref[...]) + bias[:, :HH]
    gb = dinv * (ob_ref[...] + zb_ref[...]) + bias[:, HH:]
    ga_ref[...] = ga
    gb_ref[...] = gb

    upd = jnp.concatenate([
        jnp.sum(ga, axis=0, keepdims=True),
        jnp.sum(gb, axis=0, keepdims=True),
        jnp.sum(ga * ga, axis=0, keepdims=True),
        jnp.sum(gb * gb, axis=0, keepdims=True),
        jnp.zeros((4, HH), jnp.float32),
    ], axis=0)

    @pl.when(pl.program_id(0) == 0)
    def _():
        st_ref[...] = jnp.zeros((8, HH), jnp.float32)

    st_ref[...] += upd


def _tce(oa, ob, za, zb, degA, degB, bias):
    return pl.pallas_call(
        _tce_body,
        grid=(NBLK,),
        in_specs=[
            pl.BlockSpec((BLK, HH), lambda i: (i, 0)),
            pl.BlockSpec((BLK, HH), lambda i: (i, 0)),
            pl.BlockSpec((BLK, HH), lambda i: (i, 0)),
            pl.BlockSpec((BLK, HH), lambda i: (i, 0)),
            pl.BlockSpec((BLK, 1), lambda i: (i, 0)),
            pl.BlockSpec((BLK, 1), lambda i: (i, 0)),
            pl.BlockSpec((1, H), lambda i: (0, 0)),
        ],
        out_specs=[
            pl.BlockSpec((BLK, HH), lambda i: (i, 0)),
            pl.BlockSpec((BLK, HH), lambda i: (i, 0)),
            pl.BlockSpec((8, HH), lambda i: (0, 0)),
        ],
        out_shape=[
            jax.ShapeDtypeStruct((N, HH), jnp.float32),
            jax.ShapeDtypeStruct((N, HH), jnp.float32),
            jax.ShapeDtypeStruct((8, HH), jnp.float32),
        ],
    )(oa, ob, za, zb, degA, degB, bias)


def _bn_affine(ga, gb, st, gamma, beta):
    ma = st[0:1, :] * (1.0 / N)
    mb = st[1:2, :] * (1.0 / N)
    va = st[2:3, :] * (1.0 / N) - ma * ma
    vb = st[3:4, :] * (1.0 / N) - mb * mb
    ia = 1.0 / jnp.sqrt(va + EPS)
    ib = 1.0 / jnp.sqrt(vb + EPS)
    ha = jnp.maximum((ga - ma) * ia * gamma[:, :HH] + beta[:, :HH], 0.0)
    hb = jnp.maximum((gb - mb) * ib * gamma[:, HH:] + beta[:, HH:], 0.0)
    return ha, hb


def _tck_body(ga_ref, gb_ref, st_ref, gam_ref, bet_ref, w_ref, dA_ref, dB_ref,
              za_ref, zb_ref):
    ha, hb = _bn_affine(ga_ref[...], gb_ref[...], st_ref[...],
                        gam_ref[...], bet_ref[...])
    h = jnp.concatenate([ha, hb], axis=1)
    dinv = _dinv(dA_ref[...], dB_ref[...])
    z = jnp.dot(h, w_ref[...], preferred_element_type=jnp.float32) * dinv
    za_ref[...] = z[:, :HH]
    zb_ref[...] = z[:, HH:]


def _tck(ga, gb, st, gamma, beta, W, degA, degB):
    return pl.pallas_call(
        _tck_body,
        grid=(NBLK,),
        in_specs=[
            pl.BlockSpec((BLK, HH), lambda i: (i, 0)),
            pl.BlockSpec((BLK, HH), lambda i: (i, 0)),
            pl.BlockSpec((8, HH), lambda i: (0, 0)),
            pl.BlockSpec((1, H), lambda i: (0, 0)),
            pl.BlockSpec((1, H), lambda i: (0, 0)),
            pl.BlockSpec((H, H), lambda i: (0, 0)),
            pl.BlockSpec((BLK, 1), lambda i: (i, 0)),
            pl.BlockSpec((BLK, 1), lambda i: (i, 0)),
        ],
        out_specs=[
            pl.BlockSpec((BLK, HH), lambda i: (i, 0)),
            pl.BlockSpec((BLK, HH), lambda i: (i, 0)),
        ],
        out_shape=[
            jax.ShapeDtypeStruct((N, HH), jnp.float32),
            jax.ShapeDtypeStruct((N, HH), jnp.float32),
        ],
    )(ga, gb, st, gamma, beta, W, degA, degB)


def _head_body(ga_ref, gb_ref, st_ref, gam_ref, bet_ref, batch_ref,
               fc1w_ref, fc1b_ref, g1_ref, b1_ref,
               fc2w_ref, fc2b_ref, g2_ref, b2_ref,
               fcow_ref, fcob_ref, out_ref, acc_ref):
    ha, hb = _bn_affine(ga_ref[...], gb_ref[...], st_ref[...],
                        gam_ref[...], bet_ref[...])
    ones = jnp.ones((BLK, 1), jnp.float32)
    zeros = jnp.zeros((BLK, 128 - H - 1), jnp.float32)
    h_aug = jnp.concatenate([ha, hb, ones, zeros], axis=1)
    gids = lax.broadcasted_iota(jnp.int32, (1, G), 1)
    onehot = (batch_ref[...] == gids).astype(jnp.float32)
    psum = lax.dot_general(onehot, h_aug, (((0,), (0,)), ((), ())),
                           preferred_element_type=jnp.float32,
                           precision=lax.Precision.HIGHEST)

    @pl.when(pl.program_id(0) == 0)
    def _():
        acc_ref[...] = jnp.zeros((G, 128), jnp.float32)

    acc_ref[...] += psum

    @pl.when(pl.program_id(0) == NBLK - 1)
    def _():
        acc = acc_ref[...]
        pooled = acc[:, :H] / jnp.maximum(acc[:, H:H + 1], 1.0)

        def bn_small(v, g, b):
            m = jnp.mean(v, axis=0, keepdims=True)
            var = jnp.mean((v - m) * (v - m), axis=0, keepdims=True)
            return (v - m) / jnp.sqrt(var + EPS) * g + b

        t1 = jnp.dot(pooled, fc1w_ref[...],
                     preferred_element_type=jnp.float32) + fc1b_ref[...]
        t1 = jnp.maximum(bn_small(t1, g1_ref[...], b1_ref[...]), 0.0)
        t2 = jnp.dot(t1, fc2w_ref[...],
                     preferred_element_type=jnp.float32) + fc2b_ref[...]
        t2 = jnp.maximum(bn_small(t2, g2_ref[...], b2_ref[...]), 0.0)
        out_ref[...] = jnp.dot(t2, fcow_ref[...],
                               preferred_element_type=jnp.float32) + fcob_ref[...]


def _head(ga, gb, st, gamma, beta, batch2d, p):
    full = lambda shape: pl.BlockSpec(shape, lambda i: tuple(0 for _ in shape))
    return pl.pallas_call(
        _head_body,
        grid=(NBLK,),
        in_specs=[
            pl.BlockSpec((BLK, HH), lambda i: (i, 0)),
            pl.BlockSpec((BLK, HH), lambda i: (i, 0)),
            full((8, HH)),
            full((1, H)),
            full((1, H)),
            pl.BlockSpec((BLK, 1), lambda i: (i, 0)),
            full((H, H)),
            full((1, H)),
            full((1, H)),
            full((1, H)),
            full((H, HH)),
            full((1, HH)),
            full((1, HH)),
            full((1, HH)),
            full((HH, 2)),
            full((1, 2)),
        ],
        out_specs=pl.BlockSpec((G, 2), lambda i: (0, 0)),
        out_shape=jax.ShapeDtypeStruct((G, 2), jnp.float32),
        scratch_shapes=[pltpu.VMEM((G, 128), jnp.float32)],
    )(ga, gb, st, gamma, beta, batch2d,
      p['fc1_W'], p['fc1_b'].reshape(1, H),
      p['bnf1_g'].reshape(1, H), p['bnf1_b'].reshape(1, H),
      p['fc2_W'], p['fc2_b'].reshape(1, HH),
      p['bnf2_g'].reshape(1, HH), p['bnf2_b'].reshape(1, HH),
      p['fco_W'], p['fco_b'].reshape(1, 2))


def kernel(x, edge_index, edge_weight, batch, params):
    p = params
    src = edge_index[0]
    dst = edge_index[1]
    wbits = lax.bitcast_convert_type(edge_weight, jnp.int32)
    edges = jnp.stack([
        src.reshape(NCHUNKS, CHUNK),
        dst.reshape(NCHUNKS, CHUNK),
        wbits.reshape(NCHUNKS, CHUNK),
    ], axis=1)

    deg0, deg1 = _deg_call()(edges)
    degA = deg0.reshape(N, 1)
    degB = deg1.reshape(N, 1)

    za, zb = _tck0(x, p['conv0_W'], degA, degB)
    oa, ob = _msg_call()(edges, za, zb)
    ga, gb, st = _tce(oa, ob, za, zb, degA, degB, p['conv0_b'].reshape(1, H))

    for i in (1, 2):
        za, zb = _tck(ga, gb, st,
                      p['bn%d_g' % (i - 1)].reshape(1, H),
                      p['bn%d_b' % (i - 1)].reshape(1, H),
                      p['conv%d_W' % i], degA, degB)
        oa, ob = _msg_call()(edges, za, zb)
        ga, gb, st = _tce(oa, ob, za, zb, degA, degB,
                          p['conv%d_b' % i].reshape(1, H))

    return _head(ga, gb, st,
                 p['bn2_g'].reshape(1, H), p['bn2_b'].reshape(1, H),
                 batch.reshape(N, 1), p)
